# baseline (speedup 1.0000x reference)
import functools

import jax
import jax.numpy as jnp
from jax import lax
from jax.experimental import pallas as pl
from jax.experimental.pallas import tpu as pltpu
from jax.experimental.pallas import tpu_sc as plsc

_BATCH, _SEQ, _EMBED = 4096, 200, 128
_TOTAL = _BATCH * _SEQ
_NC, _NS = 2, 16
_NW = _NC * _NS
_PER_W = _TOTAL // _NW
_CHUNK = 64
_NCHUNK = _PER_W // _CHUNK
_NBUF = 5
_LEAD = 3

assert _PER_W % _CHUNK == 0 and _NCHUNK % _NBUF == 0 and 2 <= _LEAD <= _NBUF - 2

_mesh = plsc.VectorSubcoreMesh(core_axis_name="c", subcore_axis_name="s")


@functools.partial(
    pl.kernel,
    mesh=_mesh,
    out_type=jax.ShapeDtypeStruct((_TOTAL, _EMBED), jnp.float32),
    scratch_types=[
        pltpu.VMEM((_NCHUNK, _CHUNK), jnp.int32),
        pltpu.VMEM((_NBUF, _CHUNK, _EMBED), jnp.float32),
    ] + [pltpu.SemaphoreType.DMA] * (2 * _NBUF),
)
def _gather_rows(table_hbm, idx_hbm, out_hbm, idx_v, rows_v, *sems):
    gsem = sems[:_NBUF]
    osem = sems[_NBUF:]
    c = lax.axis_index("c")
    s = lax.axis_index("s")
    wid = s * _NC + c
    base = wid * _PER_W
    pltpu.sync_copy(idx_hbm.at[wid], idx_v)

    def start_gather(k, b):
        pltpu.async_copy(table_hbm.at[idx_v.at[k]], rows_v.at[b], gsem[b])

    def wait_chunk(sem):
        pltpu.make_async_copy(rows_v.at[0],
                              out_hbm.at[pl.ds(0, _CHUNK)], sem).wait()

    for b in range(_LEAD):
        start_gather(b, b)

    def outer(i, carry):
        j0 = i * _NBUF
        for b in range(_NBUF):
            k = j0 + b
            wait_chunk(gsem[b])
            pltpu.async_copy(rows_v.at[b],
                             out_hbm.at[pl.ds(base + k * _CHUNK, _CHUNK)],
                             osem[b])
            kk = k + _LEAD
            bb = (b + _LEAD) % _NBUF

            @pl.when(jnp.logical_and(kk < _NCHUNK, kk >= _NBUF))
            def _():
                wait_chunk(osem[bb])

            @pl.when(kk < _NCHUNK)
            def _():
                start_gather(kk, bb)
        return carry

    lax.fori_loop(0, _NCHUNK // _NBUF, outer, 0)

    for b in range(_NBUF):
        wait_chunk(osem[b])


def kernel(x, table):
    idx = x.reshape(_NW, _NCHUNK, _CHUNK).astype(jnp.int32)
    out = _gather_rows(table, idx)
    return out.reshape(_BATCH, _SEQ, _EMBED)


# speedup vs baseline: 1.0022x; 1.0022x over previous
import functools

import jax
import jax.numpy as jnp
from jax import lax
from jax.experimental import pallas as pl
from jax.experimental.pallas import tpu as pltpu
from jax.experimental.pallas import tpu_sc as plsc

_BATCH, _SEQ, _EMBED = 4096, 200, 128
_TOTAL = _BATCH * _SEQ
_NC, _NS = 2, 16
_NW = _NC * _NS
_PER_W = _TOTAL // _NW
_CHUNK = 128
_NCHUNK = _PER_W // _CHUNK
_NBUF = 5
_LEAD = 3

assert _PER_W % _CHUNK == 0 and _NCHUNK % _NBUF == 0 and 2 <= _LEAD <= _NBUF - 2

_mesh = plsc.VectorSubcoreMesh(core_axis_name="c", subcore_axis_name="s")


@functools.partial(
    pl.kernel,
    mesh=_mesh,
    out_type=jax.ShapeDtypeStruct((_TOTAL, _EMBED), jnp.float32),
    scratch_types=[
        pltpu.VMEM((_NCHUNK, _CHUNK), jnp.int32),
        pltpu.VMEM((_NBUF, _CHUNK, _EMBED), jnp.float32),
    ] + [pltpu.SemaphoreType.DMA] * (2 * _NBUF),
)
def _gather_rows(table_hbm, idx_hbm, out_hbm, idx_v, rows_v, *sems):
    gsem = sems[:_NBUF]
    osem = sems[_NBUF:]
    c = lax.axis_index("c")
    s = lax.axis_index("s")
    wid = s * _NC + c
    base = wid * _PER_W
    pltpu.sync_copy(idx_hbm.at[wid], idx_v)

    def start_gather(k, b):
        pltpu.async_copy(table_hbm.at[idx_v.at[k]], rows_v.at[b], gsem[b])

    def wait_chunk(sem):
        pltpu.make_async_copy(rows_v.at[0],
                              out_hbm.at[pl.ds(0, _CHUNK)], sem).wait()

    for b in range(_LEAD):
        start_gather(b, b)

    def outer(i, carry):
        j0 = i * _NBUF
        for b in range(_NBUF):
            k = j0 + b
            wait_chunk(gsem[b])
            pltpu.async_copy(rows_v.at[b],
                             out_hbm.at[pl.ds(base + k * _CHUNK, _CHUNK)],
                             osem[b])
            kk = k + _LEAD
            bb = (b + _LEAD) % _NBUF

            @pl.when(jnp.logical_and(kk < _NCHUNK, kk >= _NBUF))
            def _():
                wait_chunk(osem[bb])

            @pl.when(kk < _NCHUNK)
            def _():
                start_gather(kk, bb)
        return carry

    lax.fori_loop(0, _NCHUNK // _NBUF, outer, 0)

    for b in range(_NBUF):
        wait_chunk(osem[b])


def kernel(x, table):
    idx = x.reshape(_NW, _NCHUNK, _CHUNK).astype(jnp.int32)
    out = _gather_rows(table, idx)
    return out.reshape(_BATCH, _SEQ, _EMBED)


# speedup vs baseline: 1.0035x; 1.0013x over previous
import functools

import jax
import jax.numpy as jnp
from jax import lax
from jax.experimental import pallas as pl
from jax.experimental.pallas import tpu as pltpu
from jax.experimental.pallas import tpu_sc as plsc

_BATCH, _SEQ, _EMBED = 4096, 200, 128
_TOTAL = _BATCH * _SEQ
_NC, _NS = 2, 16
_NW = _NC * _NS
_PER_W = _TOTAL // _NW
_CHUNK = 64
_NCHUNK = _PER_W // _CHUNK
_NBUF = 8
_LEAD = 5

assert _PER_W % _CHUNK == 0 and _NCHUNK % _NBUF == 0 and 2 <= _LEAD <= _NBUF - 2

_mesh = plsc.VectorSubcoreMesh(core_axis_name="c", subcore_axis_name="s")


@functools.partial(
    pl.kernel,
    mesh=_mesh,
    out_type=jax.ShapeDtypeStruct((_TOTAL, _EMBED), jnp.float32),
    scratch_types=[
        pltpu.VMEM((_NCHUNK, _CHUNK), jnp.int32),
        pltpu.VMEM((_NBUF, _CHUNK, _EMBED), jnp.float32),
    ] + [pltpu.SemaphoreType.DMA] * (2 * _NBUF),
)
def _gather_rows(table_hbm, idx_hbm, out_hbm, idx_v, rows_v, *sems):
    gsem = sems[:_NBUF]
    osem = sems[_NBUF:]
    c = lax.axis_index("c")
    s = lax.axis_index("s")
    wid = s * _NC + c
    base = wid * _PER_W
    pltpu.sync_copy(idx_hbm.at[wid], idx_v)

    def start_gather(k, b):
        pltpu.async_copy(table_hbm.at[idx_v.at[k]], rows_v.at[b], gsem[b])

    def wait_chunk(sem):
        pltpu.make_async_copy(rows_v.at[0],
                              out_hbm.at[pl.ds(0, _CHUNK)], sem).wait()

    for b in range(_LEAD):
        start_gather(b, b)

    def outer(i, carry):
        j0 = i * _NBUF
        for b in range(_NBUF):
            k = j0 + b
            wait_chunk(gsem[b])
            pltpu.async_copy(rows_v.at[b],
                             out_hbm.at[pl.ds(base + k * _CHUNK, _CHUNK)],
                             osem[b])
            kk = k + _LEAD
            bb = (b + _LEAD) % _NBUF

            @pl.when(jnp.logical_and(kk < _NCHUNK, kk >= _NBUF))
            def _():
                wait_chunk(osem[bb])

            @pl.when(kk < _NCHUNK)
            def _():
                start_gather(kk, bb)
        return carry

    lax.fori_loop(0, _NCHUNK // _NBUF, outer, 0)

    for b in range(_NBUF):
        wait_chunk(osem[b])


def kernel(x, table):
    idx = x.reshape(_NW, _NCHUNK, _CHUNK).astype(jnp.int32)
    out = _gather_rows(table, idx)
    return out.reshape(_BATCH, _SEQ, _EMBED)
